# baseline (speedup 1.0000x reference)
import functools

import jax
import jax.numpy as jnp
import numpy as np
from jax import lax
from jax.experimental import pallas as pl
from jax.experimental.pallas import tpu as pltpu
from jax.experimental.pallas import tpu_sc as plsc

_EMBED = 64
_LANES = 16
_NC, _NS = 2, 16
_NW = _NC * _NS
_BLK = 128
_H = 50
_B = 16384
_NBI = _B // _BLK
_NBLOCK = _H * _NBI
_BPW = _NBLOCK // _NW
_IPW = _BPW * _BLK
_PITCH = 136
_ROWS = _H * 8 * _NBI


@functools.lru_cache(maxsize=None)
def _build(vocab: int, scale: float):
    mesh = plsc.VectorSubcoreMesh(core_axis_name="c", subcore_axis_name="s")

    @functools.partial(
        pl.kernel,
        mesh=mesh,
        out_type=jax.ShapeDtypeStruct((_ROWS, 8, 128), jnp.float32),
        scratch_types=[
            pltpu.VMEM((_IPW,), jnp.int32),
            pltpu.VMEM((_BLK, _EMBED), jnp.float32),
            pltpu.VMEM((_BLK, _EMBED), jnp.float32),
            pltpu.VMEM((_EMBED, _PITCH), jnp.float32),
            pltpu.VMEM((_EMBED, _PITCH), jnp.float32),
            pltpu.SemaphoreType.DMA,
            pltpu.SemaphoreType.DMA,
            pltpu.SemaphoreType.DMA,
            pltpu.SemaphoreType.DMA,
        ],
        compiler_params=pltpu.CompilerParams(
            use_tc_tiling_on_sc=False, needs_layout_passes=False),
    )
    def emb(x_hbm, tab_hbm, out_hbm, idx_v, rows0, rows1, buft0, buft1,
            gsem0, gsem1, osem0, osem1):
        wid = lax.axis_index("s") * _NC + lax.axis_index("c")
        tid0 = wid * _BPW
        pltpu.sync_copy(x_hbm.at[pl.ds(tid0 * _BLK, _IPW)], idx_v)

        iota = lax.iota(jnp.int32, _LANES)
        ejs = [iota + (j * _LANES) for j in range(_EMBED // _LANES)]

        def g_desc(k, rows, gsem):
            return pltpu.make_async_copy(
                tab_hbm.at[idx_v.at[pl.ds(k * _BLK, _BLK)]], rows, gsem)

        def o_descs(k, buft, osem):
            tid = tid0 + k
            h = tid // _NBI
            bi = lax.rem(tid, _NBI)
            rbase = h * (8 * _NBI) + bi
            return [
                pltpu.make_async_copy(
                    buft.at[pl.ds(g * 8, 8), pl.ds(0, 128)],
                    out_hbm.at[rbase + g * _NBI],
                    osem,
                )
                for g in range(8)
            ]

        def transpose_scale(rows, buft):
            def body(i0, carry):
                for u in range(2):
                    i = i0 * 2 + u
                    bvec = iota * 0 + i
                    for j in range(_EMBED // _LANES):
                        v = rows[i, pl.ds(j * _LANES, _LANES)]
                        plsc.store_scatter(buft, [ejs[j], bvec], v * scale)
                return carry
            lax.fori_loop(0, _BLK // 2, body, 0)

        def section(k, rows_cur, rows_nxt, buft, gsem_cur, gsem_nxt, osem,
                    prefetch, drain):
            if prefetch:
                g_desc(k + 1, rows_nxt, gsem_nxt).start()
            g_desc(k, rows_cur, gsem_cur).wait()
            if drain is not None:
                def do_drain():
                    for cp in o_descs(k - 2, buft, osem):
                        cp.wait()
                pl.when(drain)(do_drain)
            transpose_scale(rows_cur, buft)
            for cp in o_descs(k, buft, osem):
                cp.start()

        g_desc(0, rows0, gsem0).start()

        def loop(i, carry):
            k0 = 2 * i
            k1 = k0 + 1
            section(k0, rows0, rows1, buft0, gsem0, gsem1, osem0,
                    prefetch=True, drain=(k0 >= 2))
            section(k1, rows1, rows0, buft1, gsem1, gsem0, osem1,
                    prefetch=False, drain=(k1 >= 3))
            def pf():
                g_desc(k1 + 1, rows0, gsem0).start()
            pl.when(k1 + 1 < _BPW)(pf)
            return carry

        lax.fori_loop(0, _BPW // 2, loop, 0)

        for cp in o_descs(_BPW - 2, buft0, osem0):
            cp.wait()
        for cp in o_descs(_BPW - 1, buft1, osem1):
            cp.wait()

    return emb


def kernel(x, embedding_table):
    b, h = x.shape
    vocab, embed = embedding_table.shape
    assert (b, h, embed) == (_B, _H, _EMBED)
    scale = float(np.sqrt(np.float32(embed)))
    x_flat = x.T.reshape(b * h).astype(jnp.int32)
    emb = _build(vocab, scale)
    out = emb(x_flat, embedding_table).reshape(_H, 8, _B // 128, 8, 128)
    return out.transpose(2, 4, 0, 1, 3).reshape(b, h, embed)


# speedup vs baseline: 1.0074x; 1.0074x over previous
import functools

import jax
import jax.numpy as jnp
import numpy as np
from jax import lax
from jax.experimental import pallas as pl
from jax.experimental.pallas import tpu as pltpu
from jax.experimental.pallas import tpu_sc as plsc

_EMBED = 64
_LANES = 16
_NC, _NS = 2, 16
_NW = _NC * _NS
_BLK = 128
_H = 50
_B = 16384
_NBI = _B // _BLK
_NBLOCK = _H * _NBI
_BPW = _NBLOCK // _NW
_IPW = _BPW * _BLK
_PITCH = 136
_ROWS = _H * 8 * _NBI


@functools.lru_cache(maxsize=None)
def _build(vocab: int, scale: float):
    mesh = plsc.VectorSubcoreMesh(core_axis_name="c", subcore_axis_name="s")

    @functools.partial(
        pl.kernel,
        mesh=mesh,
        out_type=jax.ShapeDtypeStruct((_ROWS, 8, 128), jnp.float32),
        scratch_types=[
            pltpu.VMEM((_IPW,), jnp.int32),
            pltpu.VMEM((_BLK, _EMBED), jnp.float32),
            pltpu.VMEM((_BLK, _EMBED), jnp.float32),
            pltpu.VMEM((_EMBED, _PITCH), jnp.float32),
            pltpu.VMEM((_EMBED, _PITCH), jnp.float32),
            pltpu.SemaphoreType.DMA,
            pltpu.SemaphoreType.DMA,
            pltpu.SemaphoreType.DMA,
            pltpu.SemaphoreType.DMA,
        ],
        compiler_params=pltpu.CompilerParams(
            use_tc_tiling_on_sc=False, needs_layout_passes=False,
            disable_bounds_checks=True, disable_semaphore_checks=True),
    )
    def emb(x_hbm, tab_hbm, out_hbm, idx_v, rows0, rows1, buft0, buft1,
            gsem0, gsem1, osem0, osem1):
        wid = lax.axis_index("s") * _NC + lax.axis_index("c")
        tid0 = wid * _BPW
        pltpu.sync_copy(x_hbm.at[pl.ds(tid0 * _BLK, _IPW)], idx_v)

        iota = lax.iota(jnp.int32, _LANES)
        ejs = [iota + (j * _LANES) for j in range(_EMBED // _LANES)]

        def g_desc(k, rows, gsem):
            return pltpu.make_async_copy(
                tab_hbm.at[idx_v.at[pl.ds(k * _BLK, _BLK)]], rows, gsem)

        def o_descs(k, buft, osem):
            tid = tid0 + k
            h = tid // _NBI
            bi = lax.rem(tid, _NBI)
            rbase = h * (8 * _NBI) + bi
            return [
                pltpu.make_async_copy(
                    buft.at[pl.ds(g * 8, 8), pl.ds(0, 128)],
                    out_hbm.at[rbase + g * _NBI],
                    osem,
                )
                for g in range(8)
            ]

        def transpose_scale(rows, buft):
            def body(i0, carry):
                for u in range(8):
                    i = i0 * 8 + u
                    bvec = iota * 0 + i
                    for j in range(_EMBED // _LANES):
                        v = rows[i, pl.ds(j * _LANES, _LANES)]
                        plsc.store_scatter(buft, [ejs[j], bvec], v * scale)
                return carry
            lax.fori_loop(0, _BLK // 8, body, 0)

        def section(k, rows_cur, rows_nxt, buft, gsem_cur, gsem_nxt, osem,
                    prefetch, drain):
            if prefetch:
                g_desc(k + 1, rows_nxt, gsem_nxt).start()
            g_desc(k, rows_cur, gsem_cur).wait()
            if drain is not None:
                def do_drain():
                    for cp in o_descs(k - 2, buft, osem):
                        cp.wait()
                pl.when(drain)(do_drain)
            transpose_scale(rows_cur, buft)
            for cp in o_descs(k, buft, osem):
                cp.start()

        g_desc(0, rows0, gsem0).start()

        def loop(i, carry):
            k0 = 2 * i
            k1 = k0 + 1
            section(k0, rows0, rows1, buft0, gsem0, gsem1, osem0,
                    prefetch=True, drain=(k0 >= 2))
            section(k1, rows1, rows0, buft1, gsem1, gsem0, osem1,
                    prefetch=False, drain=(k1 >= 3))
            def pf():
                g_desc(k1 + 1, rows0, gsem0).start()
            pl.when(k1 + 1 < _BPW)(pf)
            return carry

        lax.fori_loop(0, _BPW // 2, loop, 0)

        for cp in o_descs(_BPW - 2, buft0, osem0):
            cp.wait()
        for cp in o_descs(_BPW - 1, buft1, osem1):
            cp.wait()

    return emb


def kernel(x, embedding_table):
    b, h = x.shape
    vocab, embed = embedding_table.shape
    assert (b, h, embed) == (_B, _H, _EMBED)
    scale = float(np.sqrt(np.float32(embed)))
    x_flat = x.T.reshape(b * h).astype(jnp.int32)
    emb = _build(vocab, scale)
    out = emb(x_flat, embedding_table).reshape(_H, 8, _B // 128, 8, 128)
    return out.transpose(2, 4, 0, 1, 3).reshape(b, h, embed)


# speedup vs baseline: 1.4645x; 1.4537x over previous
import functools

import jax
import jax.numpy as jnp
import numpy as np
from jax import lax
from jax.experimental import pallas as pl
from jax.experimental.pallas import tpu as pltpu
from jax.experimental.pallas import tpu_sc as plsc

_EMBED = 64
_LANES = 16
_NC, _NS = 2, 16
_NW = _NC * _NS
_BLK = 128
_H = 50
_B = 16384
_NBI = _B // _BLK
_NBLOCK = _H * _NBI
_BPW = _NBLOCK // _NW
_IPW = _BPW * _BLK
_PITCH = 136
_ROWS = _H * 8 * _NBI


@functools.lru_cache(maxsize=None)
def _build(vocab: int, scale: float):
    mesh = plsc.VectorSubcoreMesh(core_axis_name="c", subcore_axis_name="s")

    @functools.partial(
        pl.kernel,
        mesh=mesh,
        out_type=jax.ShapeDtypeStruct((_ROWS, 8, 128), jnp.float32),
        scratch_types=[
            pltpu.VMEM((_IPW,), jnp.int32),
            pltpu.VMEM((_BLK, _EMBED), jnp.float32),
            pltpu.VMEM((_BLK, _EMBED), jnp.float32),
            pltpu.VMEM((_EMBED, _PITCH), jnp.float32),
            pltpu.VMEM((_EMBED, _PITCH), jnp.float32),
            pltpu.SemaphoreType.DMA,
            pltpu.SemaphoreType.DMA,
            pltpu.SemaphoreType.DMA,
            pltpu.SemaphoreType.DMA,
        ],
        compiler_params=pltpu.CompilerParams(
            use_tc_tiling_on_sc=False, needs_layout_passes=False),
    )
    def emb(x_hbm, tab_hbm, out_hbm, idx_v, rows0, rows1, buft0, buft1,
            gsem0, gsem1, osem0, osem1):
        wid = lax.axis_index("s") * _NC + lax.axis_index("c")
        tid0 = wid * _BPW
        pltpu.sync_copy(x_hbm.at[pl.ds(tid0 * _BLK, _IPW)], idx_v)

        iota = lax.iota(jnp.int32, _LANES)
        ejs = [iota + (j * _LANES) for j in range(_EMBED // _LANES)]

        def g_desc(k, rows, gsem):
            return pltpu.make_async_copy(
                tab_hbm.at[idx_v.at[pl.ds(k * _BLK, _BLK)]], rows, gsem)

        def o_descs(k, buft, osem):
            tid = tid0 + k
            h = tid // _NBI
            bi = lax.rem(tid, _NBI)
            rbase = h * (8 * _NBI) + bi
            return [
                pltpu.make_async_copy(
                    buft.at[pl.ds(g * 8, 8), pl.ds(0, 128)],
                    out_hbm.at[rbase + g * _NBI],
                    osem,
                )
                for g in range(8)
            ]

        def transpose_scale(rows, buft):
            @plsc.parallel_loop(0, _BLK, unroll=8)
            def body(i):
                bvec = iota * 0 + i
                for j in range(_EMBED // _LANES):
                    v = rows[i, pl.ds(j * _LANES, _LANES)]
                    plsc.store_scatter(buft, [ejs[j], bvec], v * scale)

        def section(k, rows_cur, rows_nxt, buft, gsem_cur, gsem_nxt, osem,
                    prefetch, drain):
            if prefetch:
                g_desc(k + 1, rows_nxt, gsem_nxt).start()
            g_desc(k, rows_cur, gsem_cur).wait()
            if drain is not None:
                def do_drain():
                    for cp in o_descs(k - 2, buft, osem):
                        cp.wait()
                pl.when(drain)(do_drain)
            transpose_scale(rows_cur, buft)
            for cp in o_descs(k, buft, osem):
                cp.start()

        g_desc(0, rows0, gsem0).start()

        def loop(i, carry):
            k0 = 2 * i
            k1 = k0 + 1
            section(k0, rows0, rows1, buft0, gsem0, gsem1, osem0,
                    prefetch=True, drain=(k0 >= 2))
            section(k1, rows1, rows0, buft1, gsem1, gsem0, osem1,
                    prefetch=False, drain=(k1 >= 3))
            def pf():
                g_desc(k1 + 1, rows0, gsem0).start()
            pl.when(k1 + 1 < _BPW)(pf)
            return carry

        lax.fori_loop(0, _BPW // 2, loop, 0)

        for cp in o_descs(_BPW - 2, buft0, osem0):
            cp.wait()
        for cp in o_descs(_BPW - 1, buft1, osem1):
            cp.wait()

    return emb


def kernel(x, embedding_table):
    b, h = x.shape
    vocab, embed = embedding_table.shape
    assert (b, h, embed) == (_B, _H, _EMBED)
    scale = float(np.sqrt(np.float32(embed)))
    x_flat = x.T.reshape(b * h).astype(jnp.int32)
    emb = _build(vocab, scale)
    out = emb(x_flat, embedding_table).reshape(_H, 8, _B // 128, 8, 128)
    return out.transpose(2, 4, 0, 1, 3).reshape(b, h, embed)


# speedup vs baseline: 1.4707x; 1.0042x over previous
import functools

import jax
import jax.numpy as jnp
import numpy as np
from jax import lax
from jax.experimental import pallas as pl
from jax.experimental.pallas import tpu as pltpu
from jax.experimental.pallas import tpu_sc as plsc

_EMBED = 64
_LANES = 16
_NC, _NS = 2, 16
_NW = _NC * _NS
_BLK = 128
_H = 50
_B = 16384
_NBI = _B // _BLK
_NBLOCK = _H * _NBI
_BPW = _NBLOCK // _NW
_IPW = _BPW * _BLK
_PITCH = 136
_ROWS = _H * 8 * _NBI


@functools.lru_cache(maxsize=None)
def _build(vocab: int, scale: float):
    mesh = plsc.VectorSubcoreMesh(core_axis_name="c", subcore_axis_name="s")

    @functools.partial(
        pl.kernel,
        mesh=mesh,
        out_type=jax.ShapeDtypeStruct((_ROWS, 8, 128), jnp.float32),
        scratch_types=[
            pltpu.VMEM((_BLK,), jnp.int32),
            pltpu.VMEM((_BLK,), jnp.int32),
            pltpu.VMEM((_BLK, _EMBED), jnp.float32),
            pltpu.VMEM((_BLK, _EMBED), jnp.float32),
            pltpu.VMEM((_EMBED, _PITCH), jnp.float32),
            pltpu.VMEM((_EMBED, _PITCH), jnp.float32),
            pltpu.SemaphoreType.DMA,
            pltpu.SemaphoreType.DMA,
            pltpu.SemaphoreType.DMA,
            pltpu.SemaphoreType.DMA,
            pltpu.SemaphoreType.DMA,
            pltpu.SemaphoreType.DMA,
        ],
        compiler_params=pltpu.CompilerParams(
            use_tc_tiling_on_sc=False, needs_layout_passes=False),
    )
    def emb(x_hbm, tab_hbm, out_hbm, idxb0, idxb1, rows0, rows1,
            buft0, buft1, isem0, isem1, gsem0, gsem1, osem0, osem1):
        wid = lax.axis_index("s") * _NC + lax.axis_index("c")
        tid0 = wid * _BPW
        idxbs = (idxb0, idxb1)
        isems = (isem0, isem1)

        def i_desc(k, par):
            return pltpu.make_async_copy(
                x_hbm.at[pl.ds((tid0 + k) * _BLK, _BLK)],
                idxbs[par], isems[par])

        iota = lax.iota(jnp.int32, _LANES)
        ejs = [iota + (j * _LANES) for j in range(_EMBED // _LANES)]

        def g_desc(par, rows, gsem):
            return pltpu.make_async_copy(
                tab_hbm.at[idxbs[par].at[pl.ds(0, _BLK)]], rows, gsem)

        def o_descs(k, buft, osem):
            tid = tid0 + k
            h = tid // _NBI
            bi = lax.rem(tid, _NBI)
            rbase = h * (8 * _NBI) + bi
            return [
                pltpu.make_async_copy(
                    buft.at[pl.ds(g * 8, 8), pl.ds(0, 128)],
                    out_hbm.at[rbase + g * _NBI],
                    osem,
                )
                for g in range(8)
            ]

        def transpose_scale(rows, buft):
            @plsc.parallel_loop(0, _BLK, unroll=8)
            def body(i):
                bvec = iota * 0 + i
                for j in range(_EMBED // _LANES):
                    v = rows[i, pl.ds(j * _LANES, _LANES)]
                    plsc.store_scatter(buft, [ejs[j], bvec], v * scale)

        def section(k, par, rows_cur, rows_nxt, buft, gsem_cur, gsem_nxt,
                    osem, prefetch, drain):
            if prefetch:
                i_desc(k + 1, 1 - par).wait()
                g_desc(1 - par, rows_nxt, gsem_nxt).start()
            g_desc(par, rows_cur, gsem_cur).wait()
            def pf_idx():
                i_desc(k + 2, par).start()
            pl.when(k + 2 < _BPW)(pf_idx)
            if drain is not None:
                def do_drain():
                    for cp in o_descs(k - 2, buft, osem):
                        cp.wait()
                pl.when(drain)(do_drain)
            transpose_scale(rows_cur, buft)
            for cp in o_descs(k, buft, osem):
                cp.start()

        i_desc(0, 0).start()
        i_desc(1, 1).start()
        i_desc(0, 0).wait()
        g_desc(0, rows0, gsem0).start()

        def loop(i, carry):
            k0 = 2 * i
            k1 = k0 + 1
            section(k0, 0, rows0, rows1, buft0, gsem0, gsem1, osem0,
                    prefetch=True, drain=(k0 >= 2))
            section(k1, 1, rows1, rows0, buft1, gsem1, gsem0, osem1,
                    prefetch=False, drain=(k1 >= 3))
            def pf():
                i_desc(k1 + 1, 0).wait()
                g_desc(0, rows0, gsem0).start()
            pl.when(k1 + 1 < _BPW)(pf)
            return carry

        lax.fori_loop(0, _BPW // 2, loop, 0)

        for cp in o_descs(_BPW - 2, buft0, osem0):
            cp.wait()
        for cp in o_descs(_BPW - 1, buft1, osem1):
            cp.wait()

    return emb


def kernel(x, embedding_table):
    b, h = x.shape
    vocab, embed = embedding_table.shape
    assert (b, h, embed) == (_B, _H, _EMBED)
    scale = float(np.sqrt(np.float32(embed)))
    x_flat = x.T.reshape(b * h).astype(jnp.int32)
    emb = _build(vocab, scale)
    out = emb(x_flat, embedding_table).reshape(_H, 8, _B // 128, 8, 128)
    return out.transpose(2, 4, 0, 1, 3).reshape(b, h, embed)
